# Initial kernel scaffold; baseline (speedup 1.0000x reference)
import functools

import jax
import jax.numpy as jnp
from jax import lax
from jax.experimental import pallas as pl
from jax.experimental.pallas import tpu as pltpu
from jax.experimental.pallas import tpu_sc as plsc


def _a1_body(x_ref, wint_ref, bin_ref, aall_ref, xp_ref, p_ref):
    xb = x_ref[0]
    xp_ref[0] = jnp.dot(xb, wint_ref[...],
                        preferred_element_type=jnp.float32) + bin_ref[0]
    p_ref[0] = jnp.dot(xb, aall_ref[...], preferred_element_type=jnp.float32)


def _stage_a1(x, W_in_T, b_in2, A_all, n, l, c, pcols, blk):
    grid = (n, l // blk)
    return pl.pallas_call(
        _a1_body,
        grid=grid,
        in_specs=[
            pl.BlockSpec((1, blk, c), lambda i, j: (i, j, 0)),
            pl.BlockSpec((c, c), lambda i, j: (0, 0)),
            pl.BlockSpec((1, c), lambda i, j: (0, 0)),
            pl.BlockSpec((c, pcols), lambda i, j: (0, 0)),
        ],
        out_specs=[
            pl.BlockSpec((1, blk, c), lambda i, j: (i, j, 0)),
            pl.BlockSpec((1, blk, pcols), lambda i, j: (i, j, 0)),
        ],
        out_shape=[
            jax.ShapeDtypeStruct((n, l, c), jnp.float32),
            jax.ShapeDtypeStruct((n, l, pcols), jnp.float32),
        ],
    )(x, W_in_T, b_in2, A_all)


def _a2_body(ppad_ref, bom_ref, idx_ref, wgt_ref, *, l, g, k):
    gk = g * k
    om = (ppad_ref[0, 0:l, 0:gk]
          + ppad_ref[0, 1:l + 1, gk:2 * gk]
          + ppad_ref[0, 2:l + 2, 2 * gk:3 * gk]
          + bom_ref[0])
    off = om[:, 0:gk]
    msk = om[:, gk:2 * gk]
    lpos = lax.broadcasted_iota(jnp.float32, (l, gk), 0)
    col = lax.broadcasted_iota(jnp.int32, (l, gk), 1)
    dil = (col % k).astype(jnp.float32) - 1.0
    loc = jnp.remainder(lpos + dil + off, float(l))
    x0 = jnp.floor(loc)
    w1 = loc - x0
    i0 = x0.astype(jnp.int32)
    i1 = i0 + 1
    v1 = i1 <= l - 1
    wa = msk * (1.0 - w1)
    wb = jnp.where(v1, msk * w1, 0.0)
    i1c = jnp.minimum(i1, l - 1)
    nidx = pl.program_id(0)
    gcol = col // k
    row_lo = (nidx * l + i0) * g + gcol
    row_hi = (nidx * l + i1c) * g + gcol
    idx_ref[0] = jnp.concatenate([row_lo, row_hi], axis=1)
    wgt_ref[0] = jnp.concatenate([wa, wb], axis=1)


def _stage_a2(Ppad, b_om2, n, l, g, k, pcols):
    gk2 = 2 * g * k
    body = functools.partial(_a2_body, l=l, g=g, k=k)
    return pl.pallas_call(
        body,
        grid=(n,),
        in_specs=[
            pl.BlockSpec((1, l + 2, pcols), lambda i: (i, 0, 0)),
            pl.BlockSpec((1, gk2 // 2), lambda i: (0, 0)),
        ],
        out_specs=[
            pl.BlockSpec((1, l, gk2), lambda i: (i, 0, 0)),
            pl.BlockSpec((1, l, gk2), lambda i: (i, 0, 0)),
        ],
        out_shape=[
            jax.ShapeDtypeStruct((n, l, gk2), jnp.int32),
            jax.ShapeDtypeStruct((n, l, gk2), jnp.float32),
        ],
    )(Ppad, b_om2)


def _make_sc_gather(tok, c, g, k, gc, nw, tpw, tb):
    nch = tpw // tb
    gk = g * k
    nv = gc // 16
    mesh = plsc.VectorSubcoreMesh(core_axis_name="c", subcore_axis_name="s")

    @functools.partial(
        pl.kernel,
        mesh=mesh,
        out_type=jax.ShapeDtypeStruct((tok, c), jnp.float32),
        scratch_types=[
            pltpu.VMEM((tb, 2 * gk), jnp.int32),
            pltpu.VMEM((tb, 2 * gk), jnp.float32),
            pltpu.VMEM((tb, 2 * gk, gc), jnp.float32),
            pltpu.VMEM((tb, c), jnp.float32),
            pltpu.SemaphoreType.DMA,
        ],
    )
    def sc_gather(table_hbm, idx_hbm, wgt_hbm, out_hbm,
                  idx_v, wgt_v, rows_v, out_v, sem):
        wid = lax.axis_index("s") * 2 + lax.axis_index("c")

        def chunk_body(ch, _):
            tok0 = wid * tpw + ch * tb
            pltpu.sync_copy(idx_hbm.at[pl.ds(tok0, tb)], idx_v)
            pltpu.sync_copy(wgt_hbm.at[pl.ds(tok0, tb)], wgt_v)

            def fire(t, _):
                pltpu.make_async_copy(
                    table_hbm.at[idx_v.at[t]], rows_v.at[t], sem).start()
                return 0

            lax.fori_loop(0, tb, fire, 0)

            def drain(t, _):
                pltpu.make_async_copy(
                    table_hbm.at[idx_v.at[t]], rows_v.at[t], sem).wait()
                return 0

            lax.fori_loop(0, tb, drain, 0)

            def tok_body(t, _):
                for gg in range(g):
                    for cc in range(nv):
                        acc = jnp.zeros((16,), jnp.float32)
                        for kk in range(k):
                            j = gg * k + kk
                            wa = wgt_v[t, j]
                            wb = wgt_v[t, gk + j]
                            acc = (acc
                                   + wa * rows_v[t, j, pl.ds(cc * 16, 16)]
                                   + wb * rows_v[t, gk + j, pl.ds(cc * 16, 16)])
                        out_v[t, pl.ds(gg * gc + cc * 16, 16)] = acc
                return 0

            lax.fori_loop(0, tb, tok_body, 0)
            pltpu.sync_copy(out_v, out_hbm.at[pl.ds(tok0, tb)])
            return 0

        lax.fori_loop(0, nch, chunk_body, 0)

    return sc_gather


def _c_body(s_ref, woutt_ref, bout_ref, out_ref):
    out_ref[...] = jnp.dot(s_ref[...], woutt_ref[...],
                           preferred_element_type=jnp.float32) + bout_ref[0]


def _stage_c(sampled, W_out_T, b_out2, tok, c, blk):
    return pl.pallas_call(
        _c_body,
        grid=(tok // blk,),
        in_specs=[
            pl.BlockSpec((blk, c), lambda i: (i, 0)),
            pl.BlockSpec((c, c), lambda i: (0, 0)),
            pl.BlockSpec((1, c), lambda i: (0, 0)),
        ],
        out_specs=pl.BlockSpec((blk, c), lambda i: (i, 0)),
        out_shape=jax.ShapeDtypeStruct((tok, c), jnp.float32),
    )(sampled, W_out_T, b_out2)


def kernel(x, W_in, b_in, dw_w, dw_b, W_red, b_red, W_off, b_off,
           W_mask, b_mask, W_out, b_out):
    n, l, c = x.shape
    k = dw_w.shape[2]
    gk = W_off.shape[0]
    g = gk // k
    gc = c // g
    tok = n * l

    Wom = jnp.concatenate([W_off, W_mask], axis=0)
    A_ks = [Wom @ (W_red * dw_w[:, 0, kk][None, :]) for kk in range(k)]
    A_all = jnp.concatenate([a.T for a in A_ks], axis=1)
    pcols = 256
    A_all = jnp.pad(A_all, ((0, 0), (0, pcols - A_all.shape[1])))
    b_om = jnp.concatenate([b_off, b_mask]) + Wom @ (W_red @ dw_b + b_red)

    xp, P = _stage_a1(x, W_in.T, b_in[None, :], A_all, n, l, c, pcols, 512)

    Ppad = jnp.pad(P, ((0, 0), (1, 1), (0, 0)))
    idx, wgt = _stage_a2(Ppad, b_om[None, :], n, l, g, k, pcols)

    table = xp.reshape(tok * g, gc)
    nw = 32
    tpw = tok // nw
    tb = 16
    sc = _make_sc_gather(tok, c, g, k, gc, nw, tpw, tb)
    sampled = sc(table, idx.reshape(tok, 2 * gk), wgt.reshape(tok, 2 * gk))

    out = _stage_c(sampled, W_out.T, b_out[None, :], tok, c, 512)
    return out.reshape(n, l, c)


# speedup vs baseline: 28.6880x; 28.6880x over previous
import functools

import jax
import jax.numpy as jnp
from jax import lax
from jax.experimental import pallas as pl
from jax.experimental.pallas import tpu as pltpu
from jax.experimental.pallas import tpu_sc as plsc


def _a1_body(x_ref, wint_ref, bin_ref, aall_ref, xp_ref, p_ref):
    xb = x_ref[0]
    xp_ref[0] = jnp.dot(xb, wint_ref[...],
                        preferred_element_type=jnp.float32) + bin_ref[0]
    p_ref[0] = jnp.dot(xb, aall_ref[...], preferred_element_type=jnp.float32)


def _stage_a1(x, W_in_T, b_in2, A_all, n, l, c, pcols, blk):
    grid = (n, l // blk)
    return pl.pallas_call(
        _a1_body,
        grid=grid,
        in_specs=[
            pl.BlockSpec((1, blk, c), lambda i, j: (i, j, 0)),
            pl.BlockSpec((c, c), lambda i, j: (0, 0)),
            pl.BlockSpec((1, c), lambda i, j: (0, 0)),
            pl.BlockSpec((c, pcols), lambda i, j: (0, 0)),
        ],
        out_specs=[
            pl.BlockSpec((1, blk, c), lambda i, j: (i, j, 0)),
            pl.BlockSpec((1, blk, pcols), lambda i, j: (i, j, 0)),
        ],
        out_shape=[
            jax.ShapeDtypeStruct((n, l, c), jnp.float32),
            jax.ShapeDtypeStruct((n, l, pcols), jnp.float32),
        ],
    )(x, W_in_T, b_in2, A_all)


def _a2_body(ppad_ref, bom_ref, idx_ref, wgt_ref, *, l, g, k):
    s = g * k
    w = 2 * s
    om = (ppad_ref[0, 0:l, 0:w]
          + ppad_ref[0, 1:l + 1, w:2 * w]
          + ppad_ref[0, 2:l + 2, 2 * w:3 * w]
          + bom_ref[0])
    off = om[:, 0:s]
    msk = om[:, s:w]
    lpos = lax.broadcasted_iota(jnp.int32, (l, s), 0).astype(jnp.float32)
    col = lax.broadcasted_iota(jnp.int32, (l, s), 1)
    dil = (col % k).astype(jnp.float32) - 1.0
    loc = jnp.remainder(lpos + dil + off, float(l))
    x0 = jnp.floor(loc)
    w1 = loc - x0
    i0 = x0.astype(jnp.int32)
    i1 = i0 + 1
    v1 = i1 <= l - 1
    wa = msk * (1.0 - w1)
    wb = jnp.where(v1, msk * w1, 0.0)
    i1c = jnp.minimum(i1, l - 1)
    nidx = pl.program_id(0)
    gcol = col // k
    row_lo = (nidx * l + i0) * g + gcol
    row_hi = (nidx * l + i1c) * g + gcol
    idx_ref[0] = jnp.concatenate([row_lo, row_hi], axis=1)
    wgt_ref[0] = jnp.concatenate(
        [wa, wb, jnp.zeros((l, 8), jnp.float32)], axis=1)


def _stage_a2(Ppad, b_om2, n, l, g, k, pcols):
    gk2 = 2 * g * k
    body = functools.partial(_a2_body, l=l, g=g, k=k)
    return pl.pallas_call(
        body,
        grid=(n,),
        in_specs=[
            pl.BlockSpec((1, l + 2, pcols), lambda i: (i, 0, 0)),
            pl.BlockSpec((1, gk2), lambda i: (0, 0)),
        ],
        out_specs=[
            pl.BlockSpec((1, l, gk2), lambda i: (i, 0, 0)),
            pl.BlockSpec((1, l, gk2 + 8), lambda i: (i, 0, 0)),
        ],
        out_shape=[
            jax.ShapeDtypeStruct((n, l, gk2), jnp.int32),
            jax.ShapeDtypeStruct((n, l, gk2 + 8), jnp.float32),
        ],
    )(Ppad, b_om2)


def _make_sc_gather(tok, c, g, k, gc, nw, tpw, tb):
    nch = tpw // tb
    gk = g * k
    nv = gc // 16
    mesh = plsc.VectorSubcoreMesh(core_axis_name="c", subcore_axis_name="s",
                                  num_cores=2, num_subcores=16)

    @functools.partial(
        pl.kernel,
        mesh=mesh,
        compiler_params=pltpu.CompilerParams(use_tc_tiling_on_sc=False),
        out_type=jax.ShapeDtypeStruct((tok, c), jnp.float32),
        scratch_types=[
            pltpu.VMEM((tb, 2 * gk), jnp.int32),
            pltpu.VMEM((tb, 2 * gk + 8), jnp.float32),
            pltpu.VMEM((tb, 2 * gk, gc), jnp.float32),
            pltpu.VMEM((tb, c), jnp.float32),
            pltpu.SemaphoreType.DMA,
        ],
    )
    def sc_gather(table_hbm, idx_hbm, wgt_hbm, out_hbm,
                  idx_v, wgt_v, rows_v, out_v, sem):
        wid = lax.axis_index("s") * 2 + lax.axis_index("c")

        def chunk_body(ch, _):
            tok0 = wid * tpw + ch * tb
            pltpu.sync_copy(idx_hbm.at[pl.ds(tok0, tb)], idx_v)
            pltpu.sync_copy(wgt_hbm.at[pl.ds(tok0, tb)], wgt_v)

            def fire(t, _):
                pltpu.make_async_copy(
                    table_hbm.at[idx_v.at[t]], rows_v.at[t], sem).start()
                return 0

            lax.fori_loop(0, tb, fire, 0)

            def drain(t, _):
                pltpu.make_async_copy(
                    table_hbm.at[idx_v.at[t]], rows_v.at[t], sem).wait()
                return 0

            lax.fori_loop(0, tb, drain, 0)

            def tok_body(t, _):
                wv = [wgt_v[t, pl.ds(i * 16, 16)]
                      for i in range((2 * gk + 8) // 16)]
                for gg in range(g):
                    for cc in range(nv):
                        acc = jnp.zeros((16,), jnp.float32)
                        for kk in range(k):
                            j = gg * k + kk
                            wa = wv[j // 16][j % 16]
                            wb = wv[(gk + j) // 16][(gk + j) % 16]
                            acc = (acc
                                   + wa * rows_v[t, j, pl.ds(cc * 16, 16)]
                                   + wb * rows_v[t, gk + j, pl.ds(cc * 16, 16)])
                        out_v[t, pl.ds(gg * gc + cc * 16, 16)] = acc
                return 0

            lax.fori_loop(0, tb, tok_body, 0)
            pltpu.sync_copy(out_v, out_hbm.at[pl.ds(tok0, tb)])
            return 0

        lax.fori_loop(0, nch, chunk_body, 0)

    return sc_gather


def _c_body(s_ref, woutt_ref, bout_ref, out_ref):
    out_ref[...] = jnp.dot(s_ref[...], woutt_ref[...],
                           preferred_element_type=jnp.float32) + bout_ref[0]


def _stage_c(sampled, W_out_T, b_out2, tok, c, blk):
    return pl.pallas_call(
        _c_body,
        grid=(tok // blk,),
        in_specs=[
            pl.BlockSpec((blk, c), lambda i: (i, 0)),
            pl.BlockSpec((c, c), lambda i: (0, 0)),
            pl.BlockSpec((1, c), lambda i: (0, 0)),
        ],
        out_specs=pl.BlockSpec((blk, c), lambda i: (i, 0)),
        out_shape=jax.ShapeDtypeStruct((tok, c), jnp.float32),
    )(sampled, W_out_T, b_out2)


def kernel(x, W_in, b_in, dw_w, dw_b, W_red, b_red, W_off, b_off,
           W_mask, b_mask, W_out, b_out):
    n, l, c = x.shape
    k = dw_w.shape[2]
    gk = W_off.shape[0]
    g = gk // k
    gc = c // g
    tok = n * l

    Wom = jnp.concatenate([W_off, W_mask], axis=0)
    A_ks = [Wom @ (W_red * dw_w[:, 0, kk][None, :]) for kk in range(k)]
    A_all = jnp.concatenate([a.T for a in A_ks], axis=1)
    pcols = 256
    A_all = jnp.pad(A_all, ((0, 0), (0, pcols - A_all.shape[1])))
    b_om = jnp.concatenate([b_off, b_mask]) + Wom @ (W_red @ dw_b + b_red)

    xp, P = _stage_a1(x, W_in.T, b_in[None, :], A_all, n, l, c, pcols, 512)

    Ppad = jnp.pad(P, ((0, 0), (1, 1), (0, 0)))
    idx, wgt = _stage_a2(Ppad, b_om[None, :], n, l, g, k, pcols)

    table = xp.reshape(tok * g, gc)
    nw = 32
    tpw = tok // nw
    tb = 16
    sc = _make_sc_gather(tok, c, g, k, gc, nw, tpw, tb)
    sampled = sc(table, idx.reshape(tok, 2 * gk),
                 wgt.reshape(tok, 2 * gk + 8))

    out = _stage_c(sampled, W_out.T, b_out[None, :], tok, c, 512)
    return out.reshape(n, l, c)


# speedup vs baseline: 36.9992x; 1.2897x over previous
import functools

import jax
import jax.numpy as jnp
from jax import lax
from jax.experimental import pallas as pl
from jax.experimental.pallas import tpu as pltpu
from jax.experimental.pallas import tpu_sc as plsc


def _a1_body(x_ref, wint_ref, bin_ref, aall_ref, xp_ref, p_ref):
    xb = x_ref[0]
    xp_ref[0] = jnp.dot(xb, wint_ref[...],
                        preferred_element_type=jnp.float32) + bin_ref[0]
    p_ref[0] = jnp.dot(xb, aall_ref[...], preferred_element_type=jnp.float32)


def _stage_a1(x, W_in_T, b_in2, A_all, n, l, c, pcols, blk):
    grid = (n, l // blk)
    return pl.pallas_call(
        _a1_body,
        grid=grid,
        in_specs=[
            pl.BlockSpec((1, blk, c), lambda i, j: (i, j, 0)),
            pl.BlockSpec((c, c), lambda i, j: (0, 0)),
            pl.BlockSpec((1, c), lambda i, j: (0, 0)),
            pl.BlockSpec((c, pcols), lambda i, j: (0, 0)),
        ],
        out_specs=[
            pl.BlockSpec((1, blk, c), lambda i, j: (i, j, 0)),
            pl.BlockSpec((1, blk, pcols), lambda i, j: (i, j, 0)),
        ],
        out_shape=[
            jax.ShapeDtypeStruct((n, l, c), jnp.float32),
            jax.ShapeDtypeStruct((n, l, pcols), jnp.float32),
        ],
    )(x, W_in_T, b_in2, A_all)


def _a2_body(ppad_ref, bom_ref, idx_ref, wgt_ref, *, l, g, k):
    s = g * k
    w = 2 * s
    om = (ppad_ref[0, 0:l, 0:w]
          + ppad_ref[0, 1:l + 1, w:2 * w]
          + ppad_ref[0, 2:l + 2, 2 * w:3 * w]
          + bom_ref[0])
    off = om[:, 0:s]
    msk = om[:, s:w]
    lpos = lax.broadcasted_iota(jnp.int32, (l, s), 0).astype(jnp.float32)
    col = lax.broadcasted_iota(jnp.int32, (l, s), 1)
    dil = (col % k).astype(jnp.float32) - 1.0
    loc = jnp.remainder(lpos + dil + off, float(l))
    x0 = jnp.floor(loc)
    w1 = loc - x0
    i0 = x0.astype(jnp.int32)
    i1 = i0 + 1
    v1 = i1 <= l - 1
    wa = msk * (1.0 - w1)
    wb = jnp.where(v1, msk * w1, 0.0)
    i1c = jnp.minimum(i1, l - 1)
    nidx = pl.program_id(0)
    gcol = col // k
    row_lo = (nidx * l + i0) * g + gcol
    row_hi = (nidx * l + i1c) * g + gcol
    idx_ref[0] = jnp.concatenate([row_lo, row_hi], axis=1)
    wgt_ref[0] = jnp.concatenate(
        [wa, wb, jnp.zeros((l, 8), jnp.float32)], axis=1)


def _stage_a2(Ppad, b_om2, n, l, g, k, pcols):
    gk2 = 2 * g * k
    body = functools.partial(_a2_body, l=l, g=g, k=k)
    return pl.pallas_call(
        body,
        grid=(n,),
        in_specs=[
            pl.BlockSpec((1, l + 2, pcols), lambda i: (i, 0, 0)),
            pl.BlockSpec((1, gk2), lambda i: (0, 0)),
        ],
        out_specs=[
            pl.BlockSpec((1, l, gk2), lambda i: (i, 0, 0)),
            pl.BlockSpec((1, l, gk2 + 8), lambda i: (i, 0, 0)),
        ],
        out_shape=[
            jax.ShapeDtypeStruct((n, l, gk2), jnp.int32),
            jax.ShapeDtypeStruct((n, l, gk2 + 8), jnp.float32),
        ],
    )(Ppad, b_om2)


def _make_sc_gather(tok, c, g, k, gc, nw, tpw, tb):
    nch = tpw // tb
    gk = g * k
    nv = gc // 16
    mesh = plsc.VectorSubcoreMesh(core_axis_name="c", subcore_axis_name="s",
                                  num_cores=2, num_subcores=16)

    @functools.partial(
        pl.kernel,
        mesh=mesh,
        compiler_params=pltpu.CompilerParams(use_tc_tiling_on_sc=False),
        out_type=jax.ShapeDtypeStruct((tok, c), jnp.float32),
        scratch_types=[
            pltpu.VMEM((tpw, 2 * gk), jnp.int32),
            pltpu.VMEM((tpw, 2 * gk + 8), jnp.float32),
            pltpu.VMEM((2, tb, 2 * gk, gc), jnp.float32),
            pltpu.VMEM((2, tb, c), jnp.float32),
            pltpu.SemaphoreType.DMA,
            pltpu.SemaphoreType.DMA,
            pltpu.SemaphoreType.DMA,
            pltpu.SemaphoreType.DMA,
        ],
    )
    def sc_gather(table_hbm, idx_hbm, wgt_hbm, out_hbm,
                  idx_v, wgt_v, rows_v, out_v, gs0, gs1, os0, os1):
        wid = lax.axis_index("s") * 2 + lax.axis_index("c")
        gsem = (gs0, gs1)
        osem = (os0, os1)

        pltpu.sync_copy(idx_hbm.at[pl.ds(wid * tpw, tpw)], idx_v)
        pltpu.sync_copy(wgt_hbm.at[pl.ds(wid * tpw, tpw)], wgt_v)

        def fire(b, ch):
            def f(t, _):
                pltpu.make_async_copy(
                    table_hbm.at[idx_v.at[ch * tb + t]],
                    rows_v.at[b, t], gsem[b]).start()
                return 0
            lax.fori_loop(0, tb, f, 0)

        def owait(b):
            pltpu.make_async_copy(
                out_v.at[b], out_hbm.at[pl.ds(0, tb)], osem[b]).wait()

        def process(b, ch, first):
            def dr(t, _):
                pltpu.make_async_copy(
                    table_hbm.at[idx_v.at[ch * tb + t]],
                    rows_v.at[b, t], gsem[b]).wait()
                return 0
            lax.fori_loop(0, tb, dr, 0)

            @pl.when(jnp.logical_not(first))
            def _():
                owait(b)

            def tok_body(t, _):
                base = ch * tb + t
                wv = [wgt_v[base, pl.ds(i * 16, 16)]
                      for i in range((2 * gk + 8) // 16)]
                for gg in range(g):
                    for cc in range(nv):
                        acc = jnp.zeros((16,), jnp.float32)
                        for kk in range(k):
                            j = gg * k + kk
                            wa = wv[j // 16][j % 16]
                            wb = wv[(gk + j) // 16][(gk + j) % 16]
                            acc = (acc
                                   + wa * rows_v[b, t, j, pl.ds(cc * 16, 16)]
                                   + wb * rows_v[b, t, gk + j,
                                                 pl.ds(cc * 16, 16)])
                        out_v[b, t, pl.ds(gg * gc + cc * 16, 16)] = acc
                return 0

            lax.fori_loop(0, tb, tok_body, 0)
            pltpu.make_async_copy(
                out_v.at[b],
                out_hbm.at[pl.ds(wid * tpw + ch * tb, tb)], osem[b]).start()

        fire(0, 0)
        fire(1, 1)

        def pair(i, _):
            ch0 = 2 * i
            process(0, ch0, i == 0)

            @pl.when(ch0 + 2 < nch)
            def _():
                fire(0, ch0 + 2)

            process(1, ch0 + 1, i == 0)

            @pl.when(ch0 + 3 < nch)
            def _():
                fire(1, ch0 + 3)
            return 0

        lax.fori_loop(0, nch // 2, pair, 0)
        owait(0)
        owait(1)

    return sc_gather


def _c_body(s_ref, woutt_ref, bout_ref, out_ref):
    out_ref[...] = jnp.dot(s_ref[...], woutt_ref[...],
                           preferred_element_type=jnp.float32) + bout_ref[0]


def _stage_c(sampled, W_out_T, b_out2, tok, c, blk):
    return pl.pallas_call(
        _c_body,
        grid=(tok // blk,),
        in_specs=[
            pl.BlockSpec((blk, c), lambda i: (i, 0)),
            pl.BlockSpec((c, c), lambda i: (0, 0)),
            pl.BlockSpec((1, c), lambda i: (0, 0)),
        ],
        out_specs=pl.BlockSpec((blk, c), lambda i: (i, 0)),
        out_shape=jax.ShapeDtypeStruct((tok, c), jnp.float32),
    )(sampled, W_out_T, b_out2)


def kernel(x, W_in, b_in, dw_w, dw_b, W_red, b_red, W_off, b_off,
           W_mask, b_mask, W_out, b_out):
    n, l, c = x.shape
    k = dw_w.shape[2]
    gk = W_off.shape[0]
    g = gk // k
    gc = c // g
    tok = n * l

    Wom = jnp.concatenate([W_off, W_mask], axis=0)
    A_ks = [Wom @ (W_red * dw_w[:, 0, kk][None, :]) for kk in range(k)]
    A_all = jnp.concatenate([a.T for a in A_ks], axis=1)
    pcols = 256
    A_all = jnp.pad(A_all, ((0, 0), (0, pcols - A_all.shape[1])))
    b_om = jnp.concatenate([b_off, b_mask]) + Wom @ (W_red @ dw_b + b_red)

    xp, P = _stage_a1(x, W_in.T, b_in[None, :], A_all, n, l, c, pcols, 512)

    Ppad = jnp.pad(P, ((0, 0), (1, 1), (0, 0)))
    idx, wgt = _stage_a2(Ppad, b_om[None, :], n, l, g, k, pcols)

    table = xp.reshape(tok * g, gc)
    nw = 32
    tpw = tok // nw
    tb = 8
    sc = _make_sc_gather(tok, c, g, k, gc, nw, tpw, tb)
    sampled = sc(table, idx.reshape(tok, 2 * gk),
                 wgt.reshape(tok, 2 * gk + 8))

    out = _stage_c(sampled, W_out.T, b_out[None, :], tok, c, 512)
    return out.reshape(n, l, c)


# speedup vs baseline: 43.2952x; 1.1702x over previous
import functools

import jax
import jax.numpy as jnp
from jax import lax
from jax.experimental import pallas as pl
from jax.experimental.pallas import tpu as pltpu
from jax.experimental.pallas import tpu_sc as plsc


def _a1_body(x_ref, wint_ref, bin_ref, aall_ref, xp_ref, p_ref):
    xb = x_ref[0]
    xp = jnp.dot(xb.astype(jnp.bfloat16), wint_ref[...],
                 preferred_element_type=jnp.float32) + bin_ref[0]
    xp_ref[0] = xp.astype(jnp.bfloat16)
    p_ref[0] = jnp.dot(xb, aall_ref[...], preferred_element_type=jnp.float32)


def _stage_a1(x, W_in_T, b_in2, A_all, n, l, c, pcols, blk):
    grid = (n, l // blk)
    return pl.pallas_call(
        _a1_body,
        grid=grid,
        in_specs=[
            pl.BlockSpec((1, blk, c), lambda i, j: (i, j, 0)),
            pl.BlockSpec((c, c), lambda i, j: (0, 0)),
            pl.BlockSpec((1, c), lambda i, j: (0, 0)),
            pl.BlockSpec((c, pcols), lambda i, j: (0, 0)),
        ],
        out_specs=[
            pl.BlockSpec((1, blk, c), lambda i, j: (i, j, 0)),
            pl.BlockSpec((1, blk, pcols), lambda i, j: (i, j, 0)),
        ],
        out_shape=[
            jax.ShapeDtypeStruct((n, l, c), jnp.bfloat16),
            jax.ShapeDtypeStruct((n, l, pcols), jnp.float32),
        ],
    )(x, W_in_T, b_in2, A_all)


def _a2_body(ppad_ref, bom_ref, idx_ref, wgt_ref, *, l, g, k):
    s = g * k
    w = 2 * s
    om = (ppad_ref[0, 0:l, 0:w]
          + ppad_ref[0, 1:l + 1, w:2 * w]
          + ppad_ref[0, 2:l + 2, 2 * w:3 * w]
          + bom_ref[0])
    off = om[:, 0:s]
    msk = om[:, s:w]
    lpos = lax.broadcasted_iota(jnp.int32, (l, s), 0).astype(jnp.float32)
    col = lax.broadcasted_iota(jnp.int32, (l, s), 1)
    dil = (col % k).astype(jnp.float32) - 1.0
    loc = jnp.remainder(lpos + dil + off, float(l))
    x0 = jnp.floor(loc)
    w1 = loc - x0
    i0 = x0.astype(jnp.int32)
    i1 = i0 + 1
    v1 = i1 <= l - 1
    wa = msk * (1.0 - w1)
    wb = jnp.where(v1, msk * w1, 0.0)
    i1c = jnp.minimum(i1, l - 1)
    nidx = pl.program_id(0)
    gcol = col // k
    row_lo = (nidx * l + i0) * g + gcol
    row_hi = (nidx * l + i1c) * g + gcol
    idx_ref[0] = jnp.concatenate([row_lo, row_hi], axis=1)
    wgt_ref[0] = jnp.concatenate(
        [wa, wb, jnp.zeros((l, 8), jnp.float32)], axis=1)


def _stage_a2(Ppad, b_om2, n, l, g, k, pcols):
    gk2 = 2 * g * k
    body = functools.partial(_a2_body, l=l, g=g, k=k)
    return pl.pallas_call(
        body,
        grid=(n,),
        in_specs=[
            pl.BlockSpec((1, l + 2, pcols), lambda i: (i, 0, 0)),
            pl.BlockSpec((1, gk2), lambda i: (0, 0)),
        ],
        out_specs=[
            pl.BlockSpec((1, l, gk2), lambda i: (i, 0, 0)),
            pl.BlockSpec((1, l, gk2 + 8), lambda i: (i, 0, 0)),
        ],
        out_shape=[
            jax.ShapeDtypeStruct((n, l, gk2), jnp.int32),
            jax.ShapeDtypeStruct((n, l, gk2 + 8), jnp.float32),
        ],
    )(Ppad, b_om2)


def _make_sc_gather(tok, c, g, k, gc, nw, tpw, tb):
    nch = tpw // tb
    gk = g * k
    nv = gc // 16
    mesh = plsc.VectorSubcoreMesh(core_axis_name="c", subcore_axis_name="s",
                                  num_cores=2, num_subcores=16)

    @functools.partial(
        pl.kernel,
        mesh=mesh,
        compiler_params=pltpu.CompilerParams(use_tc_tiling_on_sc=False,
                                             needs_layout_passes=False),
        out_type=jax.ShapeDtypeStruct((tok, c), jnp.float32),
        scratch_types=[
            pltpu.VMEM((tpw, 2 * gk), jnp.int32),
            pltpu.VMEM((tpw, 2 * gk + 8), jnp.float32),
            pltpu.VMEM((2, tb, 2 * gk, gc), jnp.bfloat16),
            pltpu.VMEM((2, tb, c), jnp.float32),
            pltpu.SemaphoreType.DMA,
            pltpu.SemaphoreType.DMA,
            pltpu.SemaphoreType.DMA,
            pltpu.SemaphoreType.DMA,
        ],
    )
    def sc_gather(table_hbm, idx_hbm, wgt_hbm, out_hbm,
                  idx_v, wgt_v, rows_v, out_v, gs0, gs1, os0, os1):
        wid = lax.axis_index("s") * 2 + lax.axis_index("c")
        gsem = (gs0, gs1)
        osem = (os0, os1)

        pltpu.sync_copy(idx_hbm.at[pl.ds(wid * tpw, tpw)], idx_v)
        pltpu.sync_copy(wgt_hbm.at[pl.ds(wid * tpw, tpw)], wgt_v)

        def fire(b, ch):
            def f(t, _):
                pltpu.make_async_copy(
                    table_hbm.at[idx_v.at[ch * tb + t]],
                    rows_v.at[b, t], gsem[b]).start()
                return 0
            lax.fori_loop(0, tb, f, 0)

        def owait(b):
            pltpu.make_async_copy(
                out_v.at[b], out_hbm.at[pl.ds(0, tb)], osem[b]).wait()

        def process(b, ch, first):
            def dr(t, _):
                pltpu.make_async_copy(
                    table_hbm.at[idx_v.at[ch * tb + t]],
                    rows_v.at[b, t], gsem[b]).wait()
                return 0
            lax.fori_loop(0, tb, dr, 0)

            @pl.when(jnp.logical_not(first))
            def _():
                owait(b)

            def tok_body(t, _):
                base = ch * tb + t
                wv = [wgt_v[base, pl.ds(i * 16, 16)]
                      for i in range((2 * gk + 8) // 16)]
                himask = jnp.full((16,), -65536, jnp.int32)
                for gg in range(g):
                    for h in range(gc // 32):
                        acc_e = jnp.zeros((16,), jnp.float32)
                        acc_o = jnp.zeros((16,), jnp.float32)
                        for kk in range(k):
                            j = gg * k + kk
                            wa = wv[j // 16][j % 16]
                            wb = wv[(gk + j) // 16][(gk + j) % 16]
                            qa = plsc.bitcast(
                                rows_v[b, t, j, pl.ds(h * 32, 32)], jnp.int32)
                            qb = plsc.bitcast(
                                rows_v[b, t, gk + j, pl.ds(h * 32, 32)],
                                jnp.int32)
                            ae = plsc.bitcast(qa << 16, jnp.float32)
                            ao = plsc.bitcast(qa & himask, jnp.float32)
                            be = plsc.bitcast(qb << 16, jnp.float32)
                            bo = plsc.bitcast(qb & himask, jnp.float32)
                            acc_e = acc_e + wa * ae + wb * be
                            acc_o = acc_o + wa * ao + wb * bo
                        out_v[b, t, pl.ds(gg * gc + h * 32, 16)] = acc_e
                        out_v[b, t, pl.ds(gg * gc + h * 32 + 16, 16)] = acc_o
                return 0

            lax.fori_loop(0, tb, tok_body, 0)
            pltpu.make_async_copy(
                out_v.at[b],
                out_hbm.at[pl.ds(wid * tpw + ch * tb, tb)], osem[b]).start()

        fire(0, 0)
        fire(1, 1)

        def pair(i, _):
            ch0 = 2 * i
            process(0, ch0, i == 0)

            @pl.when(ch0 + 2 < nch)
            def _():
                fire(0, ch0 + 2)

            process(1, ch0 + 1, i == 0)

            @pl.when(ch0 + 3 < nch)
            def _():
                fire(1, ch0 + 3)
            return 0

        lax.fori_loop(0, nch // 2, pair, 0)
        owait(0)
        owait(1)

    return sc_gather


def _c_body(s_ref, woutt_ref, bout_ref, out_ref):
    out_ref[...] = jnp.dot(s_ref[...].astype(jnp.bfloat16), woutt_ref[...],
                           preferred_element_type=jnp.float32) + bout_ref[0]


def _stage_c(sampled, W_out_T, b_out2, tok, c, blk):
    return pl.pallas_call(
        _c_body,
        grid=(tok // blk,),
        in_specs=[
            pl.BlockSpec((blk, c), lambda i: (i, 0)),
            pl.BlockSpec((c, c), lambda i: (0, 0)),
            pl.BlockSpec((1, c), lambda i: (0, 0)),
        ],
        out_specs=pl.BlockSpec((blk, c), lambda i: (i, 0)),
        out_shape=jax.ShapeDtypeStruct((tok, c), jnp.float32),
    )(sampled, W_out_T, b_out2)


def kernel(x, W_in, b_in, dw_w, dw_b, W_red, b_red, W_off, b_off,
           W_mask, b_mask, W_out, b_out):
    n, l, c = x.shape
    k = dw_w.shape[2]
    gk = W_off.shape[0]
    g = gk // k
    gc = c // g
    tok = n * l

    Wom = jnp.concatenate([W_off, W_mask], axis=0)
    A_ks = [Wom @ (W_red * dw_w[:, 0, kk][None, :]) for kk in range(k)]
    A_all = jnp.concatenate([a.T for a in A_ks], axis=1)
    pcols = 256
    A_all = jnp.pad(A_all, ((0, 0), (0, pcols - A_all.shape[1])))
    b_om = jnp.concatenate([b_off, b_mask]) + Wom @ (W_red @ dw_b + b_red)

    xp, P = _stage_a1(x, W_in.T.astype(jnp.bfloat16), b_in[None, :], A_all,
                      n, l, c, pcols, 512)

    Ppad = jnp.pad(P, ((0, 0), (1, 1), (0, 0)))
    idx, wgt = _stage_a2(Ppad, b_om[None, :], n, l, g, k, pcols)

    table = xp.reshape(tok * g, gc)
    nw = 32
    tpw = tok // nw
    tb = 8
    sc = _make_sc_gather(tok, c, g, k, gc, nw, tpw, tb)
    sampled = sc(table, idx.reshape(tok, 2 * gk),
                 wgt.reshape(tok, 2 * gk + 8))

    within = jnp.concatenate([jnp.arange(0, 32, 2), jnp.arange(1, 32, 2)])
    perm = (jnp.arange(0, c, 32)[:, None] + within[None, :]).reshape(c)
    W_out_T = W_out.T[perm].astype(jnp.bfloat16)
    out = _stage_c(sampled, W_out_T, b_out[None, :], tok, c, 512)
    return out.reshape(n, l, c)
